# baseline (speedup 1.0000x reference)
import functools

import jax
import jax.numpy as jnp
from jax import lax
from jax.experimental import pallas as pl
from jax.experimental.pallas import tpu as pltpu
from jax.experimental.pallas import tpu_sc as plsc

_B, _C, _N = 2, 4, 1024
_H = _W = 128
_G = _H * _W
_T = 2048
_NC, _NS, _L = 2, 16, 16
_NW = _NC * _NS


def _argmin_body(grd_ref, pc_ref, idx_ref):
    grd = grd_ref[...]
    pc = pc_ref[0]
    gx = grd[:, 0:1]
    gy = grd[:, 1:2]
    px = pc[0:1, :]
    py = pc[1:2, :]
    gg = gx * gx + gy * gy
    pp = px * px + py * py
    dot2 = lax.dot_general(grd, pc + pc, (((1,), (0,)), ((), ())),
                           preferred_element_type=jnp.float32)
    d2 = (gg - dot2) + pp
    lane = lax.broadcasted_iota(jnp.int32, (_T, 128), 1).astype(jnp.float32)
    bv = d2[:, 0:128]
    bi = lane
    for j in range(1, _N // 128):
        c = d2[:, j * 128:(j + 1) * 128]
        lt = c < bv
        bi = jnp.where(lt, lane + jnp.float32(j * 128), bi)
        bv = jnp.where(lt, c, bv)
    m = jnp.min(bv, axis=1, keepdims=True)
    bi_m = jnp.where(bv == m, bi, jnp.float32(_N))
    idx = jnp.min(bi_m, axis=1, keepdims=True).astype(jnp.int32)
    idx_ref[0] = idx


def _tc_argmin(grd, pc, gh):
    return pl.pallas_call(
        _argmin_body,
        grid=(_B, gh // _T),
        in_specs=[
            pl.BlockSpec((_T, 2), lambda b, t: (t, 0)),
            pl.BlockSpec((1, 2, _N), lambda b, t: (b, 0, 0)),
        ],
        out_specs=pl.BlockSpec((1, _T, 1), lambda b, t: (b, t, 0)),
        out_shape=jax.ShapeDtypeStruct((_B, gh, 1), jnp.int32),
    )(grd, pc)


@functools.cache
def _sc_gather_kernel(gh):
    chunk = (_B * gh) // _NW
    mesh = plsc.VectorSubcoreMesh(core_axis_name="c", subcore_axis_name="s")

    @functools.partial(
        pl.kernel,
        mesh=mesh,
        compiler_params=pltpu.CompilerParams(
            needs_layout_passes=False,
            skip_device_barrier=True,
            disable_bounds_checks=True,
            disable_semaphore_checks=True,
        ),
        out_type=jax.ShapeDtypeStruct((_B, _C, gh), jnp.float32),
        scratch_types=[
            pltpu.VMEM((_C * _N,), jnp.float32),
            pltpu.VMEM((chunk,), jnp.int32),
            pltpu.VMEM((_C * chunk,), jnp.float32),
        ],
    )
    def _sc_gather(r_hbm, idx_hbm, out_hbm, tbl_v, idx_v, out_v):
        wid = lax.axis_index("s") * _NC + lax.axis_index("c")
        b = wid // (_NW // _B)
        base = (wid % (_NW // _B)) * chunk
        pltpu.sync_copy(r_hbm.at[b], tbl_v)
        pltpu.sync_copy(idx_hbm.at[b, pl.ds(base, chunk)], idx_v)
        for j in range(chunk // _L):
            iv = idx_v[pl.ds(j * _L, _L)]
            for c in range(_C):
                vals = plsc.load_gather(tbl_v, [iv + c * _N])
                out_v[pl.ds(c * chunk + j * _L, _L)] = vals
        for c in range(_C):
            pltpu.sync_copy(out_v.at[pl.ds(c * chunk, chunk)],
                            out_hbm.at[b, c, pl.ds(base, chunk)])

    return _sc_gather


def kernel(R_pc, XY_pc, XY_grd):
    gh = _G // 2
    grd = XY_grd.reshape(_G, 2)
    rf = R_pc.reshape(_B, _C * _N)
    idx1 = _tc_argmin(grd[:gh], XY_pc, gh)
    out1 = _sc_gather_kernel(gh)(rf, idx1.reshape(_B, gh))
    idx2 = _tc_argmin(grd[gh:], XY_pc, gh)
    out2 = _sc_gather_kernel(gh)(rf, idx2.reshape(_B, gh))
    out = jnp.concatenate([out1, out2], axis=2)
    return out.reshape(_B, _C, _H, _W)


# speedup vs baseline: 1.1564x; 1.1564x over previous
import functools

import jax
import jax.numpy as jnp
from jax import lax
from jax.experimental import pallas as pl
from jax.experimental.pallas import tpu as pltpu
from jax.experimental.pallas import tpu_sc as plsc

_B, _C, _N = 2, 4, 1024
_H = _W = 128
_G = _H * _W
_T = 2048
_NC, _NS, _L = 2, 16, 16
_NW = _NC * _NS
_CHUNK = (_B * _G) // _NW


def _argmin_body(grd_ref, pc_ref, idx_ref):
    grd = grd_ref[...]
    pc = pc_ref[0]
    gx = grd[:, 0:1]
    gy = grd[:, 1:2]
    px = pc[0:1, :]
    py = pc[1:2, :]
    gg = gx * gx + gy * gy
    pp = px * px + py * py
    dot2 = lax.dot_general(grd, pc + pc, (((1,), (0,)), ((), ())),
                           preferred_element_type=jnp.float32)
    d2 = (gg - dot2) + pp
    lane = lax.broadcasted_iota(jnp.int32, (_T, 128), 1).astype(jnp.float32)
    bv = d2[:, 0:128]
    bi = lane
    for j in range(1, _N // 128):
        c = d2[:, j * 128:(j + 1) * 128]
        lt = c < bv
        bi = jnp.where(lt, lane + jnp.float32(j * 128), bi)
        bv = jnp.where(lt, c, bv)
    m = jnp.min(bv, axis=1, keepdims=True)
    bi_m = jnp.where(bv == m, bi, jnp.float32(_N))
    idx = jnp.min(bi_m, axis=1, keepdims=True).astype(jnp.int32)
    idx_ref[0] = idx


def _tc_argmin(grd, pc):
    return pl.pallas_call(
        _argmin_body,
        grid=(_B, _G // _T),
        in_specs=[
            pl.BlockSpec((_T, 2), lambda b, t: (t, 0)),
            pl.BlockSpec((1, 2, _N), lambda b, t: (b, 0, 0)),
        ],
        out_specs=pl.BlockSpec((1, _T, 1), lambda b, t: (b, t, 0)),
        out_shape=jax.ShapeDtypeStruct((_B, _G, 1), jnp.int32),
    )(grd, pc)


@functools.cache
def _sc_gather_kernel():
    mesh = plsc.VectorSubcoreMesh(core_axis_name="c", subcore_axis_name="s")

    @functools.partial(
        pl.kernel,
        mesh=mesh,
        compiler_params=pltpu.CompilerParams(
            needs_layout_passes=False,
            skip_device_barrier=True,
            disable_bounds_checks=True,
            disable_semaphore_checks=True,
        ),
        out_type=jax.ShapeDtypeStruct((_B, _C, _G), jnp.float32),
        scratch_types=[
            pltpu.VMEM((_C * _N,), jnp.float32),
            pltpu.VMEM((_CHUNK,), jnp.int32),
            pltpu.VMEM((_C * _CHUNK,), jnp.float32),
        ],
    )
    def _sc_gather(r_hbm, idx_hbm, out_hbm, tbl_v, idx_v, out_v):
        wid = lax.axis_index("s") * _NC + lax.axis_index("c")
        b = wid // (_NW // _B)
        base = (wid % (_NW // _B)) * _CHUNK
        pltpu.sync_copy(r_hbm.at[b], tbl_v)
        pltpu.sync_copy(idx_hbm.at[b, pl.ds(base, _CHUNK)], idx_v)
        for j in range(_CHUNK // _L):
            iv = idx_v[pl.ds(j * _L, _L)]
            for c in range(_C):
                vals = plsc.load_gather(tbl_v, [iv + c * _N])
                out_v[pl.ds(c * _CHUNK + j * _L, _L)] = vals
        for c in range(_C):
            pltpu.sync_copy(out_v.at[pl.ds(c * _CHUNK, _CHUNK)],
                            out_hbm.at[b, c, pl.ds(base, _CHUNK)])

    return _sc_gather


def kernel(R_pc, XY_pc, XY_grd):
    grd = XY_grd.reshape(_G, 2)
    idx = _tc_argmin(grd, XY_pc)
    out = _sc_gather_kernel()(R_pc.reshape(_B, _C * _N), idx.reshape(_B, _G))
    return out.reshape(_B, _C, _H, _W)


# speedup vs baseline: 1.2109x; 1.0471x over previous
import functools

import jax
import jax.numpy as jnp
from jax import lax
from jax.experimental import pallas as pl
from jax.experimental.pallas import tpu as pltpu
from jax.experimental.pallas import tpu_sc as plsc

_B, _C, _N = 2, 4, 1024
_H = _W = 128
_G = _H * _W
_T = 2048
_NC, _NS, _L = 2, 16, 16
_NW = _NC * _NS
_CHUNK = (_B * _G) // _NW


def _argmin_body(grd_ref, pc_ref, idx_ref):
    grd = grd_ref[...]
    pc = pc_ref[...]
    gx = grd[:, 0:1]
    gy = grd[:, 1:2]
    px = pc[0:1, :]
    py = pc[1:2, :]
    gg = gx * gx + gy * gy
    pp = px * px + py * py
    dot2 = lax.dot_general(grd, pc + pc, (((1,), (0,)), ((), ())),
                           preferred_element_type=jnp.float32)
    d2 = (gg - dot2) + pp
    lane = lax.broadcasted_iota(jnp.int32, (_T, 128), 1).astype(jnp.float32)
    for b in range(_B):
        bv = d2[:, b * _N:b * _N + 128]
        bi = lane
        for j in range(1, _N // 128):
            c = d2[:, b * _N + j * 128:b * _N + (j + 1) * 128]
            lt = c < bv
            bi = jnp.where(lt, lane + jnp.float32(j * 128), bi)
            bv = jnp.where(lt, c, bv)
        m = jnp.min(bv, axis=1, keepdims=True)
        bi_m = jnp.where(bv == m, bi, jnp.float32(_N))
        idx = jnp.min(bi_m, axis=1, keepdims=True).astype(jnp.int32)
        idx_ref[b] = idx


def _tc_argmin(grd, pc_cat):
    return pl.pallas_call(
        _argmin_body,
        grid=(_G // _T,),
        in_specs=[
            pl.BlockSpec((_T, 2), lambda t: (t, 0)),
            pl.BlockSpec((2, _B * _N), lambda t: (0, 0)),
        ],
        out_specs=pl.BlockSpec((_B, _T, 1), lambda t: (0, t, 0)),
        out_shape=jax.ShapeDtypeStruct((_B, _G, 1), jnp.int32),
    )(grd, pc_cat)


@functools.cache
def _sc_gather_kernel():
    mesh = plsc.VectorSubcoreMesh(core_axis_name="c", subcore_axis_name="s")

    @functools.partial(
        pl.kernel,
        mesh=mesh,
        compiler_params=pltpu.CompilerParams(
            needs_layout_passes=False,
            skip_device_barrier=True,
            disable_bounds_checks=True,
            disable_semaphore_checks=True,
        ),
        out_type=jax.ShapeDtypeStruct((_B, _C, _G), jnp.float32),
        scratch_types=[
            pltpu.VMEM((_C * _N,), jnp.float32),
            pltpu.VMEM((_CHUNK,), jnp.int32),
            pltpu.VMEM((_C * _CHUNK,), jnp.float32),
        ],
    )
    def _sc_gather(r_hbm, idx_hbm, out_hbm, tbl_v, idx_v, out_v):
        wid = lax.axis_index("s") * _NC + lax.axis_index("c")
        b = wid // (_NW // _B)
        base = (wid % (_NW // _B)) * _CHUNK
        pltpu.sync_copy(r_hbm.at[b], tbl_v)
        pltpu.sync_copy(idx_hbm.at[b, pl.ds(base, _CHUNK)], idx_v)
        for j in range(_CHUNK // _L):
            iv = idx_v[pl.ds(j * _L, _L)]
            for c in range(_C):
                vals = plsc.load_gather(tbl_v, [iv + c * _N])
                out_v[pl.ds(c * _CHUNK + j * _L, _L)] = vals
        for c in range(_C):
            pltpu.sync_copy(out_v.at[pl.ds(c * _CHUNK, _CHUNK)],
                            out_hbm.at[b, c, pl.ds(base, _CHUNK)])

    return _sc_gather


def kernel(R_pc, XY_pc, XY_grd):
    grd = XY_grd.reshape(_G, 2)
    pc_cat = jnp.transpose(XY_pc, (1, 0, 2)).reshape(2, _B * _N)
    idx = _tc_argmin(grd, pc_cat)
    out = _sc_gather_kernel()(R_pc.reshape(_B, _C * _N), idx.reshape(_B, _G))
    return out.reshape(_B, _C, _H, _W)


# speedup vs baseline: 1.2136x; 1.0023x over previous
import functools

import jax
import jax.numpy as jnp
from jax import lax
from jax.experimental import pallas as pl
from jax.experimental.pallas import tpu as pltpu
from jax.experimental.pallas import tpu_sc as plsc

_B, _C, _N = 2, 4, 1024
_H = _W = 128
_G = _H * _W
_T = 1024
_NC, _NS, _L = 2, 16, 16
_NW = _NC * _NS
_CHUNK = (_B * _G) // _NW


def _argmin_body(grd_ref, pc_ref, idx_ref):
    grd = grd_ref[...]
    pc = pc_ref[...]
    gx = grd[:, 0:1]
    gy = grd[:, 1:2]
    px = pc[0:1, :]
    py = pc[1:2, :]
    gg = gx * gx + gy * gy
    pp = px * px + py * py
    dot2 = lax.dot_general(grd, pc + pc, (((1,), (0,)), ((), ())),
                           preferred_element_type=jnp.float32)
    d2 = (gg - dot2) + pp
    lane = lax.broadcasted_iota(jnp.int32, (_T, 128), 1).astype(jnp.float32)
    for b in range(_B):
        bv = d2[:, b * _N:b * _N + 128]
        bi = lane
        for j in range(1, _N // 128):
            c = d2[:, b * _N + j * 128:b * _N + (j + 1) * 128]
            lt = c < bv
            bi = jnp.where(lt, lane + jnp.float32(j * 128), bi)
            bv = jnp.where(lt, c, bv)
        m = jnp.min(bv, axis=1, keepdims=True)
        bi_m = jnp.where(bv == m, bi, jnp.float32(_N))
        idx = jnp.min(bi_m, axis=1, keepdims=True).astype(jnp.int32)
        idx_ref[b] = idx


def _tc_argmin(grd, pc_cat):
    return pl.pallas_call(
        _argmin_body,
        grid=(_G // _T,),
        in_specs=[
            pl.BlockSpec((_T, 2), lambda t: (t, 0)),
            pl.BlockSpec((2, _B * _N), lambda t: (0, 0)),
        ],
        out_specs=pl.BlockSpec((_B, _T, 1), lambda t: (0, t, 0)),
        out_shape=jax.ShapeDtypeStruct((_B, _G, 1), jnp.int32),
    )(grd, pc_cat)


@functools.cache
def _sc_gather_kernel():
    mesh = plsc.VectorSubcoreMesh(core_axis_name="c", subcore_axis_name="s")

    @functools.partial(
        pl.kernel,
        mesh=mesh,
        compiler_params=pltpu.CompilerParams(
            needs_layout_passes=False,
            skip_device_barrier=True,
            disable_bounds_checks=True,
            disable_semaphore_checks=True,
        ),
        out_type=jax.ShapeDtypeStruct((_B, _C, _G), jnp.float32),
        scratch_types=[
            pltpu.VMEM((_C * _N,), jnp.float32),
            pltpu.VMEM((_CHUNK,), jnp.int32),
            pltpu.VMEM((_C * _CHUNK,), jnp.float32),
        ],
    )
    def _sc_gather(r_hbm, idx_hbm, out_hbm, tbl_v, idx_v, out_v):
        wid = lax.axis_index("s") * _NC + lax.axis_index("c")
        b = wid // (_NW // _B)
        base = (wid % (_NW // _B)) * _CHUNK
        pltpu.sync_copy(r_hbm.at[b], tbl_v)
        pltpu.sync_copy(idx_hbm.at[b, pl.ds(base, _CHUNK)], idx_v)
        for j in range(_CHUNK // _L):
            iv = idx_v[pl.ds(j * _L, _L)]
            for c in range(_C):
                vals = plsc.load_gather(tbl_v, [iv + c * _N])
                out_v[pl.ds(c * _CHUNK + j * _L, _L)] = vals
        for c in range(_C):
            pltpu.sync_copy(out_v.at[pl.ds(c * _CHUNK, _CHUNK)],
                            out_hbm.at[b, c, pl.ds(base, _CHUNK)])

    return _sc_gather


def kernel(R_pc, XY_pc, XY_grd):
    grd = XY_grd.reshape(_G, 2)
    pc_cat = jnp.transpose(XY_pc, (1, 0, 2)).reshape(2, _B * _N)
    idx = _tc_argmin(grd, pc_cat)
    out = _sc_gather_kernel()(R_pc.reshape(_B, _C * _N), idx.reshape(_B, _G))
    return out.reshape(_B, _C, _H, _W)
